# Initial kernel scaffold; baseline (speedup 1.0000x reference)
import jax
import jax.numpy as jnp
from jax.experimental import pallas as pl


def kernel(img_z, desc_batch_idx, role_idx, pred_i, op_i, nt_i, pu_i, t_idx, k_idx, f_idx, role_emb, str_emb, t_emb, k_emb, face_emb, W1, b1, W2, b2):
    raise NotImplementedError("write your pallas kernel here")


# speedup vs baseline: 5.6587x; 5.6587x over previous
import functools

import jax
import jax.numpy as jnp
from jax import lax
from jax.experimental import pallas as pl
from jax.experimental.pallas import tpu as pltpu
from jax.experimental.pallas import tpu_sc as plsc

M = 204800
B = 4096
H = 256
SV = 65536
NC = 2
NS = 16
NW = NC * NS
C = 64
PER_W = M // NW
CHUNKS_PER_W = PER_W // C
N_CHUNKS = M // C

TK_BLK = 2048
TK_GRID = SV // TK_BLK


def _prep(img_z, role_emb, str_emb, t_emb, k_emb, face_emb, w1i, b1):
    def body(img_ref, role_ref, str_ref, t_ref, k_ref, f_ref, w1_ref, b1_ref,
             a_img, str128, rf, tk):
        i = pl.program_id(0)
        z64 = jnp.zeros((TK_BLK, 64), jnp.float32)
        str128[...] = jnp.concatenate([str_ref[...], z64], axis=1)

        t_blk = t_ref[...]
        t_rep = jnp.broadcast_to(t_blk[:, None, :], (8, 256, 16))
        t_rep = t_rep.reshape(TK_BLK, 16)
        k_rep = jnp.broadcast_to(k_ref[...][None, :, :], (8, 256, 16))
        k_rep = k_rep.reshape(TK_BLK, 16)
        tk[...] = jnp.concatenate(
            [t_rep, k_rep, jnp.zeros((TK_BLK, 96), jnp.float32)], axis=1)

        @pl.when(i == 0)
        def _():
            a_img[...] = (jnp.dot(img_ref[...], w1_ref[...],
                                  preferred_element_type=jnp.float32,
                                  precision=lax.Precision.HIGHEST)
                          + b1_ref[...])
            role_rep = jnp.broadcast_to(role_ref[...][:, None, :], (8, 8, 16))
            role_rep = role_rep.reshape(64, 16)
            f_pad = jnp.concatenate(
                [f_ref[...], jnp.zeros((8, 8), jnp.float32)], axis=1)
            f_rep = jnp.broadcast_to(f_pad[None, :, :], (8, 8, 16))
            f_rep = f_rep.reshape(64, 16)
            rf[...] = jnp.concatenate(
                [role_rep, f_rep, jnp.zeros((64, 96), jnp.float32)], axis=1)

    full = lambda shape: pl.BlockSpec(shape, lambda i: tuple(0 for _ in shape))
    return pl.pallas_call(
        body,
        grid=(TK_GRID,),
        in_specs=[
            full((B, 256)),
            full((8, 16)),
            pl.BlockSpec((TK_BLK, 64), lambda i: (i, 0)),
            pl.BlockSpec((8, 16), lambda i: (i, 0)),
            full((256, 16)),
            full((8, 8)),
            full((256, H)),
            full((1, H)),
        ],
        out_specs=[
            full((B, H)),
            pl.BlockSpec((TK_BLK, 128), lambda i: (i, 0)),
            full((64, 128)),
            pl.BlockSpec((TK_BLK, 128), lambda i: (i, 0)),
        ],
        out_shape=[
            jax.ShapeDtypeStruct((B, H), jnp.float32),
            jax.ShapeDtypeStruct((SV, 128), jnp.float32),
            jax.ShapeDtypeStruct((64, 128), jnp.float32),
            jax.ShapeDtypeStruct((SV, 128), jnp.float32),
        ],
    )(img_z, role_emb, str_emb, t_emb, k_emb, face_emb, w1i, b1)


def _sc_gather(idxs, a_img, rf_t, str_t, tk_t):
    mesh = plsc.VectorSubcoreMesh(core_axis_name="c", subcore_axis_name="s")

    @functools.partial(
        pl.kernel,
        out_type=[
            jax.ShapeDtypeStruct((M, H), jnp.float32),
            jax.ShapeDtypeStruct((M, 128), jnp.float32),
            jax.ShapeDtypeStruct((M, 128), jnp.float32),
            jax.ShapeDtypeStruct((M, 128), jnp.float32),
            jax.ShapeDtypeStruct((M, 128), jnp.float32),
            jax.ShapeDtypeStruct((M, 128), jnp.float32),
            jax.ShapeDtypeStruct((M, 128), jnp.float32),
        ],
        mesh=mesh,
        scratch_types=[
            pltpu.VMEM((8, C), jnp.int32),
            pltpu.VMEM((C, H), jnp.float32),
            pltpu.VMEM((C, 128), jnp.float32),
            pltpu.VMEM((C, 128), jnp.float32),
            pltpu.VMEM((C, 128), jnp.float32),
            pltpu.VMEM((C, 128), jnp.float32),
            pltpu.VMEM((C, 128), jnp.float32),
            pltpu.VMEM((C, 128), jnp.float32),
            pltpu.SemaphoreType.DMA,
            pltpu.SemaphoreType.DMA,
        ],
    )
    def k(idxs_hbm, img_hbm, rf_hbm, str_hbm, tk_hbm,
          g_img, g_rf, g_pred, g_op, g_nt, g_pu, g_tk,
          ibuf, bimg, brf, bpred, bop, bnt, bpu, btk, gsem, wsem):
        wid = lax.axis_index("s") * NC + lax.axis_index("c")

        def chunk(c, carry):
            g = wid * CHUNKS_PER_W + c
            pltpu.sync_copy(idxs_hbm.at[g], ibuf)
            cps = [
                pltpu.async_copy(img_hbm.at[ibuf.at[0]], bimg, gsem),
                pltpu.async_copy(rf_hbm.at[ibuf.at[1]], brf, gsem),
                pltpu.async_copy(str_hbm.at[ibuf.at[2]], bpred, gsem),
                pltpu.async_copy(str_hbm.at[ibuf.at[3]], bop, gsem),
                pltpu.async_copy(str_hbm.at[ibuf.at[4]], bnt, gsem),
                pltpu.async_copy(str_hbm.at[ibuf.at[5]], bpu, gsem),
                pltpu.async_copy(tk_hbm.at[ibuf.at[6]], btk, gsem),
            ]
            for cp in cps:
                cp.wait()
            row = pl.ds(g * C, C)
            wps = [
                pltpu.async_copy(bimg, g_img.at[row, :], wsem),
                pltpu.async_copy(brf, g_rf.at[row, :], wsem),
                pltpu.async_copy(bpred, g_pred.at[row, :], wsem),
                pltpu.async_copy(bop, g_op.at[row, :], wsem),
                pltpu.async_copy(bnt, g_nt.at[row, :], wsem),
                pltpu.async_copy(bpu, g_pu.at[row, :], wsem),
                pltpu.async_copy(btk, g_tk.at[row, :], wsem),
            ]
            for wp in wps:
                wp.wait()
            return carry

        lax.fori_loop(0, CHUNKS_PER_W, chunk, 0)

    return k(idxs, a_img, rf_t, str_t, tk_t)


def _head(g_img, g_rf, g_pred, g_op, g_nt, g_pu, g_tk, w1rest, w2, b2):
    bm = 2048

    def body(gi_ref, rf_ref, p_ref, o_ref, n_ref, u_ref, tk_ref,
             w1_ref, w2_ref, b2_ref, out_ref):
        rest = jnp.concatenate(
            [rf_ref[...][:, :32], p_ref[...][:, :64], o_ref[...][:, :64],
             n_ref[...][:, :64], u_ref[...][:, :64], tk_ref[...][:, :32]],
            axis=1)
        h = gi_ref[...] + jnp.dot(rest, w1_ref[...],
                                  preferred_element_type=jnp.float32,
                                  precision=lax.Precision.HIGHEST)
        h = jnp.maximum(h, 0.0)
        out_ref[...] = (jnp.sum(h * w2_ref[...], axis=1, keepdims=True)
                        + b2_ref[...])

    out = pl.pallas_call(
        body,
        grid=(M // bm,),
        in_specs=[
            pl.BlockSpec((bm, H), lambda i: (i, 0)),
            pl.BlockSpec((bm, 128), lambda i: (i, 0)),
            pl.BlockSpec((bm, 128), lambda i: (i, 0)),
            pl.BlockSpec((bm, 128), lambda i: (i, 0)),
            pl.BlockSpec((bm, 128), lambda i: (i, 0)),
            pl.BlockSpec((bm, 128), lambda i: (i, 0)),
            pl.BlockSpec((bm, 128), lambda i: (i, 0)),
            pl.BlockSpec((320, H), lambda i: (0, 0)),
            pl.BlockSpec((1, H), lambda i: (0, 0)),
            pl.BlockSpec((1, 1), lambda i: (0, 0)),
        ],
        out_specs=pl.BlockSpec((bm, 1), lambda i: (i, 0)),
        out_shape=jax.ShapeDtypeStruct((M, 1), jnp.float32),
    )(g_img, g_rf, g_pred, g_op, g_nt, g_pu, g_tk, w1rest, w2, b2)
    return out[:, 0]


def kernel(img_z, desc_batch_idx, role_idx, pred_i, op_i, nt_i, pu_i,
           t_idx, k_idx, f_idx, role_emb, str_emb, t_emb, k_emb, face_emb,
           W1, b1, W2, b2):
    i32 = jnp.int32
    rf_i = role_idx.astype(i32) * 8 + f_idx.astype(i32)
    tk_i = t_idx.astype(i32) * 256 + k_idx.astype(i32)
    zeros = jnp.zeros((M,), i32)
    idxs = jnp.stack([
        desc_batch_idx.astype(i32), rf_i, pred_i.astype(i32),
        op_i.astype(i32), nt_i.astype(i32), pu_i.astype(i32), tk_i, zeros,
    ])
    idxs = idxs.reshape(8, N_CHUNKS, C).transpose(1, 0, 2)

    w1t = W1.T
    w1rest = jnp.concatenate([
        w1t[256:272],
        w1t[560:568],
        jnp.zeros((8, H), jnp.float32),
        w1t[272:528],
        w1t[528:560],
    ], axis=0)

    a_img, str_t, rf_t, tk_t = _prep(img_z, role_emb, str_emb, t_emb, k_emb,
                                     face_emb, w1t[0:256], b1.reshape(1, H))
    g = _sc_gather(idxs, a_img, rf_t, str_t, tk_t)
    return _head(*g, w1rest, W2.reshape(1, H), b2.reshape(1, 1))
